# baseline (speedup 1.0000x reference)
import jax
import jax.numpy as jnp
from jax import lax
from jax.experimental import pallas as pl
from jax.experimental.pallas import tpu as pltpu
from jax.experimental.pallas import tpu_sc as plsc

_NTOK = 16384
_D = 64
_VOCAB = 1000000
_NC = 2
_NS = 16
_NW = _NC * _NS
_PER_W = _NTOK // _NW
_CHUNK = 128
_NCHUNK = _PER_W // _CHUNK
_LANES = 16
_HPAD = 1000064
_ZCH = 62592
_ZLAST = _HPAD - 15 * _ZCH
_NMAIN = 999936
_TAIL = _VOCAB - _NMAIN
_SC_PT = 14336
_SC_COLS = _NW * _SC_PT
_TC_COLS = _NMAIN - _SC_COLS
_SC_BASE = _TC_COLS
_CC_TC = 3584
_CC_SC = 512
_NCH_SC = _SC_PT // _CC_SC


def _hist_body(ids_hbm, zeros_hbm, out_hbm, idx_v, vals_v, zbuf_v, hist_sh):
    c = lax.axis_index("c")
    s = lax.axis_index("s")
    wid = s * _NC + c

    pltpu.sync_copy(ids_hbm.at[wid], idx_v)

    for ci in range(_CHUNK // _LANES):
        vals_v[pl.ds(ci * _LANES, _LANES)] = jnp.full((_LANES,), 1.0,
                                                      jnp.float32)

    pltpu.sync_copy(zeros_hbm, zbuf_v)

    @pl.when(s < _NS - 1)
    def _zmain():
        pltpu.sync_copy(zbuf_v, hist_sh.at[pl.ds(s * _ZCH, _ZCH)])

    @pl.when(s == _NS - 1)
    def _zlast():
        pltpu.sync_copy(zbuf_v.at[pl.ds(0, _ZLAST)],
                        hist_sh.at[pl.ds(15 * _ZCH, _ZLAST)])

    plsc.subcore_barrier()

    for k in range(_NCHUNK):
        pltpu.sync_copy(vals_v, hist_sh.at[idx_v.at[k]], add=True)
    plsc.subcore_barrier()

    @pl.when(s < _NS - 1)
    def _dmain():
        pltpu.sync_copy(hist_sh.at[pl.ds(s * _ZCH, _ZCH)], zbuf_v)
        pltpu.sync_copy(zbuf_v, out_hbm.at[c, pl.ds(s * _ZCH, _ZCH)])

    @pl.when(s == _NS - 1)
    def _dlast():
        zpart = zbuf_v.at[pl.ds(0, _ZLAST)]
        pltpu.sync_copy(hist_sh.at[pl.ds(15 * _ZCH, _ZLAST)], zpart)
        pltpu.sync_copy(zpart, out_hbm.at[c, pl.ds(15 * _ZCH, _ZLAST)])


def _scscan_body(tbl_hbm, cnt_hbm, out_hbm,
                 bufs_v, cbuf_v, csum_v, acc_v, sem0, sem1):
    c = lax.axis_index("c")
    s = lax.axis_index("s")
    wid = s * _NC + c
    base = _SC_BASE + wid * _SC_PT

    zero16 = jnp.zeros((_LANES,), jnp.float32)
    for j in range(_D):
        acc_v[j, pl.ds(0, _LANES)] = zero16

    def fire(ch, slot, sem):
        col0 = base + ch * _CC_SC
        pltpu.async_copy(tbl_hbm.at[:, pl.ds(col0, _CC_SC)],
                         bufs_v.at[slot], sem)
        pltpu.async_copy(cnt_hbm.at[:, pl.ds(col0, _CC_SC)],
                         cbuf_v.at[slot], sem)

    def wait(ch, slot, sem):
        col0 = base + ch * _CC_SC
        pltpu.make_async_copy(tbl_hbm.at[:, pl.ds(col0, _CC_SC)],
                              bufs_v.at[slot], sem).wait()
        pltpu.make_async_copy(cnt_hbm.at[:, pl.ds(col0, _CC_SC)],
                              cbuf_v.at[slot], sem).wait()

    def compute(slot):
        buf = bufs_v.at[slot]
        cb = cbuf_v.at[slot]
        for v in range(_CC_SC // _LANES):
            csum_v[pl.ds(_LANES * v, _LANES)] = (
                cb[0, pl.ds(_LANES * v, _LANES)]
                + cb[1, pl.ds(_LANES * v, _LANES)]
            )

        def rowgrp(g, carry):
            j0 = g * 8
            accs = [acc_v[j0 + r, pl.ds(0, _LANES)] for r in range(8)]
            for v in range(_CC_SC // _LANES):
                cs = csum_v[pl.ds(_LANES * v, _LANES)]
                for r in range(8):
                    accs[r] = accs[r] + buf[j0 + r,
                                            pl.ds(_LANES * v, _LANES)] * cs
            for r in range(8):
                acc_v[j0 + r, pl.ds(0, _LANES)] = accs[r]
            return carry

        lax.fori_loop(0, _D // 8, rowgrp, 0)

    fire(0, 0, sem0)

    def pair(k2, carry):
        k0 = 2 * k2
        fire(k0 + 1, 1, sem1)
        wait(k0, 0, sem0)
        compute(0)

        @pl.when(k0 + 2 < _NCH_SC)
        def _next():
            fire(k0 + 2, 0, sem0)

        wait(k0 + 1, 1, sem1)
        compute(1)
        return carry

    lax.fori_loop(0, _NCH_SC // 2, pair, 0)
    pltpu.sync_copy(acc_v, out_hbm.at[wid])


def _scan_body(tbl_ref, cnt_ref, o_ref):
    i = pl.program_id(0)

    @pl.when(i == 0)
    def _init():
        o_ref[...] = jnp.zeros_like(o_ref)

    csum = cnt_ref[0, :] + cnt_ref[1, :]
    o_ref[...] += jnp.dot(
        tbl_ref[...], csum, preferred_element_type=jnp.float32
    )[None, :]


def _finish_body(main_ref, scp_ref, ctail_ref, ttail_ref, wt_ref, o_ref):
    scpart = jnp.sum(jnp.sum(scp_ref[...], axis=0), axis=1)
    ct = ctail_ref[0, :] + ctail_ref[1, :]
    tail = jnp.dot(ttail_ref[...], ct, preferred_element_type=jnp.float32)
    pooled = (main_ref[0, :] + scpart + tail) * (1.0 / _NTOK)
    o_ref[...] = jnp.dot(pooled[None, :], wt_ref[...],
                         preferred_element_type=jnp.float32)


@jax.jit
def _run(ids, emb_t, wt):
    mesh = plsc.VectorSubcoreMesh(core_axis_name="c", subcore_axis_name="s")
    counts = pl.kernel(
        _hist_body,
        out_type=jax.ShapeDtypeStruct((_NC, _HPAD), jnp.float32),
        mesh=mesh,
        scratch_types=[
            pltpu.VMEM((_NCHUNK, _CHUNK), jnp.int32),
            pltpu.VMEM((_CHUNK,), jnp.float32),
            pltpu.VMEM((_ZCH,), jnp.float32),
            pltpu.VMEM_SHARED((_HPAD,), jnp.float32),
        ],
        name="token_histogram_sc",
    )(ids, jnp.zeros((_ZCH,), jnp.float32))

    scpart = pl.kernel(
        _scscan_body,
        out_type=jax.ShapeDtypeStruct((_NW, _D, _LANES), jnp.float32),
        mesh=mesh,
        scratch_types=[
            pltpu.VMEM((2, _D, _CC_SC), jnp.float32),
            pltpu.VMEM((2, _NC, _CC_SC), jnp.float32),
            pltpu.VMEM((_CC_SC,), jnp.float32),
            pltpu.VMEM((_D, _LANES), jnp.float32),
            pltpu.SemaphoreType.DMA,
            pltpu.SemaphoreType.DMA,
        ],
        name="table_scan_matvec_sc",
    )(emb_t, counts)

    main = pl.pallas_call(
        _scan_body,
        grid=(_TC_COLS // _CC_TC,),
        in_specs=[
            pl.BlockSpec((_D, _CC_TC), lambda i: (0, i)),
            pl.BlockSpec((_NC, _CC_TC), lambda i: (0, i)),
        ],
        out_specs=pl.BlockSpec((1, _D), lambda i: (0, 0)),
        out_shape=jax.ShapeDtypeStruct((1, _D), jnp.float32),
        name="table_scan_matvec_tc",
    )(emb_t, counts)

    ctail = lax.slice(counts, (0, _NMAIN), (_NC, _VOCAB))
    ttail = lax.slice(emb_t, (0, _NMAIN), (_D, _VOCAB))
    out = pl.pallas_call(
        _finish_body,
        out_shape=jax.ShapeDtypeStruct((1, _D), jnp.float32),
        name="finish_tc",
    )(main, scpart, ctail, ttail, wt)
    return out


def kernel(token_ids, embedding, W):
    ids = token_ids.astype(jnp.int32).reshape(_NW, _NCHUNK, _CHUNK)
    return _run(ids, embedding.T, W.T)


# speedup vs baseline: 1.4527x; 1.4527x over previous
import jax
import jax.numpy as jnp
from jax import lax
from jax.experimental import pallas as pl
from jax.experimental.pallas import tpu as pltpu
from jax.experimental.pallas import tpu_sc as plsc

_NTOK = 16384
_D = 64
_VOCAB = 1000000
_NC = 2
_NS = 16
_NW = _NC * _NS
_PER_W = _NTOK // _NW
_CHUNK = 128
_NCHUNK = _PER_W // _CHUNK
_LANES = 16
_HPAD = 1000064
_ZCH = 62592
_ZLAST = _HPAD - 15 * _ZCH
_C = 32256
_NMAIN = 31 * _C
_TAIL = _VOCAB - _NMAIN


def _hist_body(ids_hbm, zeros_hbm, out_hbm, idx_v, vals_v, zbuf_v, hist_sh):
    c = lax.axis_index("c")
    s = lax.axis_index("s")
    wid = s * _NC + c

    pltpu.sync_copy(ids_hbm.at[wid], idx_v)

    for ci in range(_CHUNK // _LANES):
        vals_v[pl.ds(ci * _LANES, _LANES)] = jnp.full((_LANES,), 1.0,
                                                      jnp.float32)

    pltpu.sync_copy(zeros_hbm, zbuf_v)

    @pl.when(s < _NS - 1)
    def _zmain():
        pltpu.sync_copy(zbuf_v, hist_sh.at[pl.ds(s * _ZCH, _ZCH)])

    @pl.when(s == _NS - 1)
    def _zlast():
        pltpu.sync_copy(zbuf_v.at[pl.ds(0, _ZLAST)],
                        hist_sh.at[pl.ds(15 * _ZCH, _ZLAST)])

    plsc.subcore_barrier()

    for k in range(_NCHUNK):
        pltpu.sync_copy(vals_v, hist_sh.at[idx_v.at[k]], add=True)
    plsc.subcore_barrier()

    @pl.when(s < _NS - 1)
    def _dmain():
        pltpu.sync_copy(hist_sh.at[pl.ds(s * _ZCH, _ZCH)], zbuf_v)
        pltpu.sync_copy(zbuf_v, out_hbm.at[c, pl.ds(s * _ZCH, _ZCH)])

    @pl.when(s == _NS - 1)
    def _dlast():
        zpart = zbuf_v.at[pl.ds(0, _ZLAST)]
        pltpu.sync_copy(hist_sh.at[pl.ds(15 * _ZCH, _ZLAST)], zpart)
        pltpu.sync_copy(zpart, out_hbm.at[c, pl.ds(15 * _ZCH, _ZLAST)])


def _scan_body(tbl_ref, cnt_ref, o_ref):
    i = pl.program_id(0)

    @pl.when(i == 0)
    def _init():
        o_ref[...] = jnp.zeros_like(o_ref)

    csum = cnt_ref[0, :] + cnt_ref[1, :]
    o_ref[...] += jnp.dot(
        tbl_ref[...], csum, preferred_element_type=jnp.float32
    )[None, :]


def _finish_body(main_ref, ctail_ref, ttail_ref, wt_ref, o_ref):
    ct = ctail_ref[0, :] + ctail_ref[1, :]
    tail = jnp.dot(ttail_ref[...], ct, preferred_element_type=jnp.float32)
    pooled = (main_ref[0, :] + tail) * (1.0 / _NTOK)
    o_ref[...] = jnp.dot(pooled[None, :], wt_ref[...],
                         preferred_element_type=jnp.float32)


@jax.jit
def _run(ids, emb_t, wt):
    mesh = plsc.VectorSubcoreMesh(core_axis_name="c", subcore_axis_name="s")
    counts = pl.kernel(
        _hist_body,
        out_type=jax.ShapeDtypeStruct((_NC, _HPAD), jnp.float32),
        mesh=mesh,
        scratch_types=[
            pltpu.VMEM((_NCHUNK, _CHUNK), jnp.int32),
            pltpu.VMEM((_CHUNK,), jnp.float32),
            pltpu.VMEM((_ZCH,), jnp.float32),
            pltpu.VMEM_SHARED((_HPAD,), jnp.float32),
        ],
        name="token_histogram_sc",
    )(ids, jnp.zeros((_ZCH,), jnp.float32))

    main = pl.pallas_call(
        _scan_body,
        grid=(_NMAIN // _C,),
        in_specs=[
            pl.BlockSpec((_D, _C), lambda i: (0, i)),
            pl.BlockSpec((_NC, _C), lambda i: (0, i)),
        ],
        out_specs=pl.BlockSpec((1, _D), lambda i: (0, 0)),
        out_shape=jax.ShapeDtypeStruct((1, _D), jnp.float32),
        name="table_scan_matvec_tc",
    )(emb_t, counts)

    ctail = lax.slice(counts, (0, _NMAIN), (_NC, _VOCAB))
    ttail = lax.slice(emb_t, (0, _NMAIN), (_D, _VOCAB))
    out = pl.pallas_call(
        _finish_body,
        out_shape=jax.ShapeDtypeStruct((1, _D), jnp.float32),
        name="finish_tc",
    )(main, ctail, ttail, wt)
    return out


def kernel(token_ids, embedding, W):
    ids = token_ids.astype(jnp.int32).reshape(_NW, _NCHUNK, _CHUNK)
    return _run(ids, embedding.T, W.T)
